# baseline (speedup 1.0000x reference)
import functools

import jax
import jax.numpy as jnp
from jax import lax
from jax.experimental import pallas as pl
from jax.experimental.pallas import tpu as pltpu
from jax.experimental.pallas import tpu_sc as plsc

CHUNK = 80
BLK = 25
NUM_CORES = 2
NUM_SUBCORES = 16
NUM_TILES = NUM_CORES * NUM_SUBCORES


ROWBLK = 2000


def _mm_body(x_ref, w_ref, o_ref):
  o_ref[...] = lax.dot_general(
      x_ref[...], w_ref[...], (((1,), (0,)), ((), ())),
      precision=lax.Precision.HIGHEST, preferred_element_type=jnp.float32)


def _matmul(x, w):
  n, din = x.shape
  dout = w.shape[1]
  return pl.pallas_call(
      _mm_body,
      grid=(n // ROWBLK,),
      in_specs=[
          pl.BlockSpec((ROWBLK, din), lambda i: (i, 0)),
          pl.BlockSpec((din, dout), lambda i: (0, 0)),
      ],
      out_specs=pl.BlockSpec((ROWBLK, dout), lambda i: (i, 0)),
      out_shape=jax.ShapeDtypeStruct((n, dout), jnp.float32),
  )(x, w)


def _scale_split_body(deg_ref, mm_ref, g_ref, dis_ref):
  dis = lax.rsqrt(deg_ref[0, 0, 0, :] + deg_ref[1, 0, 0, :])
  g = dis[:, None] * mm_ref[...]
  dh = g.shape[1] // 2
  g_ref[0] = g[:, :dh]
  g_ref[1] = g[:, dh:]
  dis_ref[0, 0] = dis


def _scale_split(deg, mm):
  n, d = mm.shape
  nb = n // ROWBLK
  deg4 = deg.reshape(2, nb, 1, ROWBLK)
  return pl.pallas_call(
      _scale_split_body,
      grid=(nb,),
      in_specs=[
          pl.BlockSpec((2, 1, 1, ROWBLK), lambda i: (0, i, 0, 0)),
          pl.BlockSpec((ROWBLK, d), lambda i: (i, 0)),
      ],
      out_specs=[
          pl.BlockSpec((2, ROWBLK, d // 2), lambda i: (0, i, 0)),
          pl.BlockSpec((1, 1, ROWBLK), lambda i: (i, 0, 0)),
      ],
      out_shape=[
          jax.ShapeDtypeStruct((2, n, d // 2), jnp.float32),
          jax.ShapeDtypeStruct((nb, 1, ROWBLK), jnp.float32),
      ],
  )(deg4, mm)


def _mid_body(acc_ref, dis_ref, b1_ref, w2_ref, g_ref):
  dis = dis_ref[0, 0, :]
  acc = jnp.concatenate([acc_ref[0], acc_ref[1]], axis=1)
  h = jnp.maximum(dis[:, None] * acc + b1_ref[...][None, :], 0.0)
  g2 = lax.dot_general(
      h, w2_ref[...], (((1,), (0,)), ((), ())),
      precision=lax.Precision.HIGHEST, preferred_element_type=jnp.float32)
  g_ref[...] = dis[:, None] * g2


def _mid_dense(acc1, dis3, b1, w2):
  _, n, dh = acc1.shape
  dout = w2.shape[1]
  return pl.pallas_call(
      _mid_body,
      grid=(n // ROWBLK,),
      in_specs=[
          pl.BlockSpec((2, ROWBLK, dh), lambda i: (0, i, 0)),
          pl.BlockSpec((1, 1, ROWBLK), lambda i: (i, 0, 0)),
          pl.BlockSpec((b1.shape[0],), lambda i: (0,)),
          pl.BlockSpec(w2.shape, lambda i: (0, 0)),
      ],
      out_specs=pl.BlockSpec((ROWBLK, dout), lambda i: (i, 0)),
      out_shape=jax.ShapeDtypeStruct((n, dout), jnp.float32),
  )(acc1, dis3, b1, w2)


def _final_body(acc_ref, dis_ref, b2_ref, o_ref):
  acc = acc_ref[0] + acc_ref[1]
  o_ref[...] = dis_ref[0, 0, :][:, None] * acc + b2_ref[...][None, :]


def _final(acc2, dis3, b2):
  _, n, dh = acc2.shape
  d = b2.shape[0]
  return pl.pallas_call(
      _final_body,
      grid=(n // ROWBLK,),
      in_specs=[
          pl.BlockSpec((2, ROWBLK, dh), lambda i: (0, i, 0)),
          pl.BlockSpec((1, 1, ROWBLK), lambda i: (i, 0, 0)),
          pl.BlockSpec((d,), lambda i: (0,)),
      ],
      out_specs=pl.BlockSpec((ROWBLK, d), lambda i: (i, 0)),
      out_shape=jax.ShapeDtypeStruct((n, d), jnp.float32),
  )(acc2, dis3, b2)


def _sc_mesh():
  return plsc.VectorSubcoreMesh(core_axis_name="c", subcore_axis_name="s")


def _edge_stream(gsrc, row_blk, col_blk, n_blocks, blk_sz,
                 acc_sp, row_t, col_t, msg_v, gsem, ssem):
  @pl.loop(0, n_blocks)
  def _(blk):
    pltpu.sync_copy(row_blk(blk), row_t)
    pltpu.sync_copy(col_blk(blk), col_t)
    pltpu.async_copy(gsrc.at[row_t.at[0]], msg_v.at[0], gsem.at[0])
    pltpu.async_copy(gsrc.at[row_t.at[1]], msg_v.at[1], gsem.at[1])
    pltpu.async_copy(gsrc.at[row_t.at[2]], msg_v.at[2], gsem.at[2])

    @pl.loop(0, blk_sz)
    def _(i):
      b = lax.rem(i, 4)
      pltpu.make_async_copy(gsrc.at[row_t.at[i]], msg_v.at[b],
                            gsem.at[b]).wait()
      pltpu.async_copy(msg_v.at[b], acc_sp.at[col_t.at[i]], ssem.at[b],
                       add=True)

      @pl.when(i + 3 < blk_sz)
      def _():
        b2 = lax.rem(i + 3, 4)

        @pl.when(i > 0)
        def _():
          pltpu.make_async_copy(msg_v.at[b2], acc_sp.at[col_t.at[i]],
                                ssem.at[b2]).wait()

        pltpu.async_copy(gsrc.at[row_t.at[i + 3]], msg_v.at[b2],
                         gsem.at[b2])

    for j in (blk_sz - 4, blk_sz - 3, blk_sz - 2, blk_sz - 1):
      pltpu.make_async_copy(msg_v.at[j % 4], acc_sp.at[col_t.at[j]],
                            ssem.at[j % 4]).wait()


def _hist(idx4, init_deg, ones_chunk):
  n = init_deg.shape[1]
  n_grp = idx4.shape[1]
  grp_per_tile = n_grp // NUM_TILES
  blk_sz = idx4.shape[2]
  per_tile = grp_per_tile * blk_sz
  W = 4

  @functools.partial(
      pl.kernel,
      out_type=jax.ShapeDtypeStruct((2, n), jnp.float32),
      mesh=_sc_mesh(),
      scratch_types=[
          pltpu.VMEM_SHARED((n,), jnp.float32),
          pltpu.VMEM((grp_per_tile, blk_sz, CHUNK), jnp.int32),
          pltpu.VMEM((CHUNK,), jnp.float32),
          pltpu.SemaphoreType.DMA((W,)),
      ],
  )
  def hist_kernel(idx_hbm, init_hbm, ones_hbm, deg_hbm,
                  deg_sp, col_t, ones_v, ssem):
    c = lax.axis_index("c")
    s = lax.axis_index("s")
    tid = c * NUM_SUBCORES + s
    pltpu.sync_copy(idx_hbm.at[1, pl.ds(tid * grp_per_tile, grp_per_tile)],
                    col_t)
    pltpu.sync_copy(ones_hbm, ones_v)

    @pl.when(s == 0)
    def _():
      pltpu.sync_copy(init_hbm.at[c], deg_sp)

    plsc.subcore_barrier()

    def col_at(i):
      return col_t.at[lax.div(i, blk_sz), lax.rem(i, blk_sz)]

    @pl.loop(0, per_tile)
    def _(i):
      @pl.when(i >= W)
      def _():
        pltpu.make_async_copy(ones_v, deg_sp.at[col_at(i)],
                              ssem.at[lax.rem(i, W)]).wait()

      pltpu.async_copy(ones_v, deg_sp.at[col_at(i)],
                       ssem.at[lax.rem(i, W)], add=True)

    @pl.loop(per_tile - W, per_tile)
    def _(i):
      pltpu.make_async_copy(ones_v, deg_sp.at[col_at(i)],
                            ssem.at[lax.rem(i, W)]).wait()

    plsc.subcore_barrier()

    @pl.when(s == 0)
    def _():
      pltpu.sync_copy(deg_sp, deg_hbm.at[c])

  return hist_kernel(idx4, init_deg, ones_chunk)


def _aggregate(g, idx4):
  _, n, dh = g.shape
  n_blocks = idx4.shape[1] // NUM_SUBCORES
  blk_sz = idx4.shape[2]
  rows_lo = (n // NUM_SUBCORES) // 8 * 8
  rows_hi = n - rows_lo * (NUM_SUBCORES - 1)

  @functools.partial(
      pl.kernel,
      out_type=jax.ShapeDtypeStruct((2, n, dh), jnp.float32),
      mesh=_sc_mesh(),
      scratch_types=[
          pltpu.VMEM_SHARED((n, dh), jnp.float32),
          pltpu.VMEM((blk_sz, CHUNK), jnp.int32),
          pltpu.VMEM((blk_sz, CHUNK), jnp.int32),
          pltpu.VMEM((4, CHUNK, dh), jnp.float32),
          pltpu.SemaphoreType.DMA((4,)),
          pltpu.SemaphoreType.DMA((4,)),
      ],
  )
  def agg_kernel(g_hbm, idx_hbm, out_hbm,
                 acc_sp, row_t, col_t, msg_v, gsem, ssem):
    c = lax.axis_index("c")
    s = lax.axis_index("s")
    rbase = pl.multiple_of(s * rows_lo, 8)

    @pl.when(s < NUM_SUBCORES - 1)
    def _():
      pltpu.sync_copy(g_hbm.at[c, pl.ds(rbase, rows_lo), :],
                      acc_sp.at[pl.ds(rbase, rows_lo), :])

    @pl.when(s == NUM_SUBCORES - 1)
    def _():
      pltpu.sync_copy(g_hbm.at[c, pl.ds(rbase, rows_hi), :],
                      acc_sp.at[pl.ds(rbase, rows_hi), :])

    plsc.subcore_barrier()

    _edge_stream(g_hbm.at[c], lambda blk: idx_hbm.at[0, s * n_blocks + blk],
                 lambda blk: idx_hbm.at[1, s * n_blocks + blk],
                 n_blocks, blk_sz,
                 acc_sp, row_t, col_t, msg_v, gsem, ssem)

    plsc.subcore_barrier()

    @pl.when(s < NUM_SUBCORES - 1)
    def _():
      pltpu.sync_copy(acc_sp.at[pl.ds(rbase, rows_lo), :],
                      out_hbm.at[c, pl.ds(rbase, rows_lo), :])

    @pl.when(s == NUM_SUBCORES - 1)
    def _():
      pltpu.sync_copy(acc_sp.at[pl.ds(rbase, rows_hi), :],
                      out_hbm.at[c, pl.ds(rbase, rows_hi), :])

  return agg_kernel(g, idx4)


def _aggregate_edge_split(g, zeros_init, idx4):
  n, dh = g.shape
  n_blocks = idx4.shape[1] // NUM_TILES
  blk_sz = idx4.shape[2]
  rows_lo = (n // NUM_SUBCORES) // 8 * 8
  rows_hi = n - rows_lo * (NUM_SUBCORES - 1)

  @functools.partial(
      pl.kernel,
      out_type=jax.ShapeDtypeStruct((2, n, dh), jnp.float32),
      mesh=_sc_mesh(),
      scratch_types=[
          pltpu.VMEM_SHARED((n, dh), jnp.float32),
          pltpu.VMEM((blk_sz, CHUNK), jnp.int32),
          pltpu.VMEM((blk_sz, CHUNK), jnp.int32),
          pltpu.VMEM((4, CHUNK, dh), jnp.float32),
          pltpu.SemaphoreType.DMA((4,)),
          pltpu.SemaphoreType.DMA((4,)),
      ],
  )
  def agg_kernel(g_hbm, z_hbm, idx_hbm, out_hbm,
                 acc_sp, row_t, col_t, msg_v, gsem, ssem):
    c = lax.axis_index("c")
    s = lax.axis_index("s")
    rbase = pl.multiple_of(s * rows_lo, 8)

    def init_rows(nrows):
      @pl.when(c == 0)
      def _():
        pltpu.sync_copy(g_hbm.at[pl.ds(rbase, nrows), :],
                        acc_sp.at[pl.ds(rbase, nrows), :])

      @pl.when(c == 1)
      def _():
        pltpu.sync_copy(z_hbm.at[pl.ds(rbase, nrows), :],
                        acc_sp.at[pl.ds(rbase, nrows), :])

    @pl.when(s < NUM_SUBCORES - 1)
    def _():
      init_rows(rows_lo)

    @pl.when(s == NUM_SUBCORES - 1)
    def _():
      init_rows(rows_hi)

    plsc.subcore_barrier()

    gbase = (c * NUM_SUBCORES + s) * n_blocks
    _edge_stream(g_hbm, lambda blk: idx_hbm.at[0, gbase + blk],
                 lambda blk: idx_hbm.at[1, gbase + blk],
                 n_blocks, blk_sz,
                 acc_sp, row_t, col_t, msg_v, gsem, ssem)

    plsc.subcore_barrier()

    @pl.when(s < NUM_SUBCORES - 1)
    def _():
      pltpu.sync_copy(acc_sp.at[pl.ds(rbase, rows_lo), :],
                      out_hbm.at[c, pl.ds(rbase, rows_lo), :])

    @pl.when(s == NUM_SUBCORES - 1)
    def _():
      pltpu.sync_copy(acc_sp.at[pl.ds(rbase, rows_hi), :],
                      out_hbm.at[c, pl.ds(rbase, rows_hi), :])

  return agg_kernel(g, zeros_init, idx4)


def kernel(x, edge_index, conv1_weight, conv1_bias, conv2_weight, conv2_bias):
  n = x.shape[0]
  e = edge_index.shape[1]
  assert e % (NUM_TILES * BLK * CHUNK) == 0
  n_groups = e // (BLK * CHUNK)
  idx4 = edge_index.reshape(2, n_groups, BLK, CHUNK)
  init_deg = jnp.stack([jnp.ones((n,), jnp.float32),
                        jnp.zeros((n,), jnp.float32)])
  ones_chunk = jnp.ones((CHUNK,), jnp.float32)
  zeros_feat = jnp.zeros((n, conv2_weight.shape[1]), jnp.float32)

  mm1 = _matmul(x, conv1_weight)
  deg = _hist(idx4, init_deg, ones_chunk)
  g1, dis3 = _scale_split(deg, mm1)
  acc1 = _aggregate(g1, idx4)
  g2 = _mid_dense(acc1, dis3, conv1_bias, conv2_weight)
  acc2 = _aggregate_edge_split(g2, zeros_feat, idx4)
  return _final(acc2, dis3, conv2_bias)


# speedup vs baseline: 1.0245x; 1.0245x over previous
import functools

import jax
import jax.numpy as jnp
from jax import lax
from jax.experimental import pallas as pl
from jax.experimental.pallas import tpu as pltpu
from jax.experimental.pallas import tpu_sc as plsc

CHUNK = 80
BLK = 25
NUM_CORES = 2
NUM_SUBCORES = 16
NUM_TILES = NUM_CORES * NUM_SUBCORES


ROWBLK = 2000


def _mm_body(x_ref, w_ref, o_ref):
  o_ref[...] = lax.dot_general(
      x_ref[...], w_ref[...], (((1,), (0,)), ((), ())),
      precision=lax.Precision.HIGHEST, preferred_element_type=jnp.float32)


def _matmul(x, w):
  n, din = x.shape
  dout = w.shape[1]
  return pl.pallas_call(
      _mm_body,
      grid=(n // ROWBLK,),
      in_specs=[
          pl.BlockSpec((ROWBLK, din), lambda i: (i, 0)),
          pl.BlockSpec((din, dout), lambda i: (0, 0)),
      ],
      out_specs=pl.BlockSpec((ROWBLK, dout), lambda i: (i, 0)),
      out_shape=jax.ShapeDtypeStruct((n, dout), jnp.float32),
  )(x, w)


def _scale_split_body(deg_ref, mm_ref, g_ref, dis_ref):
  dis = lax.rsqrt(deg_ref[0, 0, 0, :] + deg_ref[1, 0, 0, :])
  g = dis[:, None] * mm_ref[...]
  dh = g.shape[1] // 2
  g_ref[0] = g[:, :dh]
  g_ref[1] = g[:, dh:]
  dis_ref[0, 0] = dis


def _scale_split(deg, mm):
  n, d = mm.shape
  nb = n // ROWBLK
  deg4 = deg.reshape(2, nb, 1, ROWBLK)
  return pl.pallas_call(
      _scale_split_body,
      grid=(nb,),
      in_specs=[
          pl.BlockSpec((2, 1, 1, ROWBLK), lambda i: (0, i, 0, 0)),
          pl.BlockSpec((ROWBLK, d), lambda i: (i, 0)),
      ],
      out_specs=[
          pl.BlockSpec((2, ROWBLK, d // 2), lambda i: (0, i, 0)),
          pl.BlockSpec((1, 1, ROWBLK), lambda i: (i, 0, 0)),
      ],
      out_shape=[
          jax.ShapeDtypeStruct((2, n, d // 2), jnp.float32),
          jax.ShapeDtypeStruct((nb, 1, ROWBLK), jnp.float32),
      ],
  )(deg4, mm)


def _mid_body(acc_ref, dis_ref, b1_ref, w2_ref, g_ref):
  dis = dis_ref[0, 0, :]
  acc = jnp.concatenate([acc_ref[0], acc_ref[1]], axis=1)
  h = jnp.maximum(dis[:, None] * acc + b1_ref[...][None, :], 0.0)
  g2 = lax.dot_general(
      h, w2_ref[...], (((1,), (0,)), ((), ())),
      precision=lax.Precision.HIGHEST, preferred_element_type=jnp.float32)
  g_ref[...] = dis[:, None] * g2


def _mid_dense(acc1, dis3, b1, w2):
  _, n, dh = acc1.shape
  dout = w2.shape[1]
  return pl.pallas_call(
      _mid_body,
      grid=(n // ROWBLK,),
      in_specs=[
          pl.BlockSpec((2, ROWBLK, dh), lambda i: (0, i, 0)),
          pl.BlockSpec((1, 1, ROWBLK), lambda i: (i, 0, 0)),
          pl.BlockSpec((b1.shape[0],), lambda i: (0,)),
          pl.BlockSpec(w2.shape, lambda i: (0, 0)),
      ],
      out_specs=pl.BlockSpec((ROWBLK, dout), lambda i: (i, 0)),
      out_shape=jax.ShapeDtypeStruct((n, dout), jnp.float32),
  )(acc1, dis3, b1, w2)


def _final_body(acc_ref, dis_ref, b2_ref, o_ref):
  acc = acc_ref[0] + acc_ref[1]
  o_ref[...] = dis_ref[0, 0, :][:, None] * acc + b2_ref[...][None, :]


def _final(acc2, dis3, b2):
  _, n, dh = acc2.shape
  d = b2.shape[0]
  return pl.pallas_call(
      _final_body,
      grid=(n // ROWBLK,),
      in_specs=[
          pl.BlockSpec((2, ROWBLK, dh), lambda i: (0, i, 0)),
          pl.BlockSpec((1, 1, ROWBLK), lambda i: (i, 0, 0)),
          pl.BlockSpec((d,), lambda i: (0,)),
      ],
      out_specs=pl.BlockSpec((ROWBLK, d), lambda i: (i, 0)),
      out_shape=jax.ShapeDtypeStruct((n, d), jnp.float32),
  )(acc2, dis3, b2)


def _sc_mesh():
  return plsc.VectorSubcoreMesh(core_axis_name="c", subcore_axis_name="s")


def _edge_stream(gsrc, row_blk, col_blk, n_blocks, blk_sz,
                 acc_sp, row_t, col_t, msg_v, gsem, ssem):
  @pl.loop(0, n_blocks)
  def _(blk):
    pltpu.sync_copy(row_blk(blk), row_t)
    pltpu.sync_copy(col_blk(blk), col_t)
    pltpu.async_copy(gsrc.at[row_t.at[0]], msg_v.at[0], gsem.at[0])
    pltpu.async_copy(gsrc.at[row_t.at[1]], msg_v.at[1], gsem.at[1])

    @pl.loop(0, blk_sz)
    def _(i):
      b = lax.rem(i, 3)
      pltpu.make_async_copy(gsrc.at[row_t.at[i]], msg_v.at[b],
                            gsem.at[b]).wait()
      pltpu.async_copy(msg_v.at[b], acc_sp.at[col_t.at[i]], ssem.at[b],
                       add=True)

      @pl.when(i + 2 < blk_sz)
      def _():
        b2 = lax.rem(i + 2, 3)

        @pl.when(i > 0)
        def _():
          pltpu.make_async_copy(msg_v.at[b2], acc_sp.at[col_t.at[i]],
                                ssem.at[b2]).wait()

        pltpu.async_copy(gsrc.at[row_t.at[i + 2]], msg_v.at[b2],
                         gsem.at[b2])

    for j in (blk_sz - 3, blk_sz - 2, blk_sz - 1):
      pltpu.make_async_copy(msg_v.at[j % 3], acc_sp.at[col_t.at[j]],
                            ssem.at[j % 3]).wait()


def _hist(idx4, init_deg, ones_chunk):
  n = init_deg.shape[1]
  n_grp = idx4.shape[1]
  grp_per_tile = n_grp // NUM_TILES
  blk_sz = idx4.shape[2]
  per_tile = grp_per_tile * blk_sz
  W = 4

  @functools.partial(
      pl.kernel,
      out_type=jax.ShapeDtypeStruct((2, n), jnp.float32),
      mesh=_sc_mesh(),
      scratch_types=[
          pltpu.VMEM_SHARED((n,), jnp.float32),
          pltpu.VMEM((grp_per_tile, blk_sz, CHUNK), jnp.int32),
          pltpu.VMEM((CHUNK,), jnp.float32),
          pltpu.SemaphoreType.DMA((W,)),
      ],
  )
  def hist_kernel(idx_hbm, init_hbm, ones_hbm, deg_hbm,
                  deg_sp, col_t, ones_v, ssem):
    c = lax.axis_index("c")
    s = lax.axis_index("s")
    tid = c * NUM_SUBCORES + s
    pltpu.sync_copy(idx_hbm.at[1, pl.ds(tid * grp_per_tile, grp_per_tile)],
                    col_t)
    pltpu.sync_copy(ones_hbm, ones_v)

    @pl.when(s == 0)
    def _():
      pltpu.sync_copy(init_hbm.at[c], deg_sp)

    plsc.subcore_barrier()

    def col_at(i):
      return col_t.at[lax.div(i, blk_sz), lax.rem(i, blk_sz)]

    @pl.loop(0, per_tile)
    def _(i):
      @pl.when(i >= W)
      def _():
        pltpu.make_async_copy(ones_v, deg_sp.at[col_at(i)],
                              ssem.at[lax.rem(i, W)]).wait()

      pltpu.async_copy(ones_v, deg_sp.at[col_at(i)],
                       ssem.at[lax.rem(i, W)], add=True)

    @pl.loop(per_tile - W, per_tile)
    def _(i):
      pltpu.make_async_copy(ones_v, deg_sp.at[col_at(i)],
                            ssem.at[lax.rem(i, W)]).wait()

    plsc.subcore_barrier()

    @pl.when(s == 0)
    def _():
      pltpu.sync_copy(deg_sp, deg_hbm.at[c])

  return hist_kernel(idx4, init_deg, ones_chunk)


def _aggregate(g, idx4):
  _, n, dh = g.shape
  n_blocks = idx4.shape[1] // NUM_SUBCORES
  blk_sz = idx4.shape[2]
  rows_lo = (n // NUM_SUBCORES) // 8 * 8
  rows_hi = n - rows_lo * (NUM_SUBCORES - 1)

  @functools.partial(
      pl.kernel,
      out_type=jax.ShapeDtypeStruct((2, n, dh), jnp.float32),
      mesh=_sc_mesh(),
      scratch_types=[
          pltpu.VMEM_SHARED((n, dh), jnp.float32),
          pltpu.VMEM((blk_sz, CHUNK), jnp.int32),
          pltpu.VMEM((blk_sz, CHUNK), jnp.int32),
          pltpu.VMEM((3, CHUNK, dh), jnp.float32),
          pltpu.SemaphoreType.DMA((3,)),
          pltpu.SemaphoreType.DMA((3,)),
      ],
  )
  def agg_kernel(g_hbm, idx_hbm, out_hbm,
                 acc_sp, row_t, col_t, msg_v, gsem, ssem):
    c = lax.axis_index("c")
    s = lax.axis_index("s")
    rbase = pl.multiple_of(s * rows_lo, 8)

    @pl.when(s < NUM_SUBCORES - 1)
    def _():
      pltpu.sync_copy(g_hbm.at[c, pl.ds(rbase, rows_lo), :],
                      acc_sp.at[pl.ds(rbase, rows_lo), :])

    @pl.when(s == NUM_SUBCORES - 1)
    def _():
      pltpu.sync_copy(g_hbm.at[c, pl.ds(rbase, rows_hi), :],
                      acc_sp.at[pl.ds(rbase, rows_hi), :])

    plsc.subcore_barrier()

    _edge_stream(g_hbm.at[c], lambda blk: idx_hbm.at[0, s * n_blocks + blk],
                 lambda blk: idx_hbm.at[1, s * n_blocks + blk],
                 n_blocks, blk_sz,
                 acc_sp, row_t, col_t, msg_v, gsem, ssem)

    plsc.subcore_barrier()

    @pl.when(s < NUM_SUBCORES - 1)
    def _():
      pltpu.sync_copy(acc_sp.at[pl.ds(rbase, rows_lo), :],
                      out_hbm.at[c, pl.ds(rbase, rows_lo), :])

    @pl.when(s == NUM_SUBCORES - 1)
    def _():
      pltpu.sync_copy(acc_sp.at[pl.ds(rbase, rows_hi), :],
                      out_hbm.at[c, pl.ds(rbase, rows_hi), :])

  return agg_kernel(g, idx4)


def _aggregate_edge_split(g, zeros_init, idx4):
  n, dh = g.shape
  n_blocks = idx4.shape[1] // NUM_TILES
  blk_sz = idx4.shape[2]
  rows_lo = (n // NUM_SUBCORES) // 8 * 8
  rows_hi = n - rows_lo * (NUM_SUBCORES - 1)

  @functools.partial(
      pl.kernel,
      out_type=jax.ShapeDtypeStruct((2, n, dh), jnp.float32),
      mesh=_sc_mesh(),
      scratch_types=[
          pltpu.VMEM_SHARED((n, dh), jnp.float32),
          pltpu.VMEM((blk_sz, CHUNK), jnp.int32),
          pltpu.VMEM((blk_sz, CHUNK), jnp.int32),
          pltpu.VMEM((3, CHUNK, dh), jnp.float32),
          pltpu.SemaphoreType.DMA((3,)),
          pltpu.SemaphoreType.DMA((3,)),
      ],
  )
  def agg_kernel(g_hbm, z_hbm, idx_hbm, out_hbm,
                 acc_sp, row_t, col_t, msg_v, gsem, ssem):
    c = lax.axis_index("c")
    s = lax.axis_index("s")
    rbase = pl.multiple_of(s * rows_lo, 8)

    def init_rows(nrows):
      @pl.when(c == 0)
      def _():
        pltpu.sync_copy(g_hbm.at[pl.ds(rbase, nrows), :],
                        acc_sp.at[pl.ds(rbase, nrows), :])

      @pl.when(c == 1)
      def _():
        pltpu.sync_copy(z_hbm.at[pl.ds(rbase, nrows), :],
                        acc_sp.at[pl.ds(rbase, nrows), :])

    @pl.when(s < NUM_SUBCORES - 1)
    def _():
      init_rows(rows_lo)

    @pl.when(s == NUM_SUBCORES - 1)
    def _():
      init_rows(rows_hi)

    plsc.subcore_barrier()

    gbase = (c * NUM_SUBCORES + s) * n_blocks
    _edge_stream(g_hbm, lambda blk: idx_hbm.at[0, gbase + blk],
                 lambda blk: idx_hbm.at[1, gbase + blk],
                 n_blocks, blk_sz,
                 acc_sp, row_t, col_t, msg_v, gsem, ssem)

    plsc.subcore_barrier()

    @pl.when(s < NUM_SUBCORES - 1)
    def _():
      pltpu.sync_copy(acc_sp.at[pl.ds(rbase, rows_lo), :],
                      out_hbm.at[c, pl.ds(rbase, rows_lo), :])

    @pl.when(s == NUM_SUBCORES - 1)
    def _():
      pltpu.sync_copy(acc_sp.at[pl.ds(rbase, rows_hi), :],
                      out_hbm.at[c, pl.ds(rbase, rows_hi), :])

  return agg_kernel(g, zeros_init, idx4)


def kernel(x, edge_index, conv1_weight, conv1_bias, conv2_weight, conv2_bias):
  n = x.shape[0]
  e = edge_index.shape[1]
  assert e % (NUM_TILES * BLK * CHUNK) == 0
  n_groups = e // (BLK * CHUNK)
  idx4 = edge_index.reshape(2, n_groups, BLK, CHUNK)
  init_deg = jnp.stack([jnp.ones((n,), jnp.float32),
                        jnp.zeros((n,), jnp.float32)])
  ones_chunk = jnp.ones((CHUNK,), jnp.float32)
  zeros_feat = jnp.zeros((n, conv2_weight.shape[1]), jnp.float32)

  mm1 = _matmul(x, conv1_weight)
  deg = _hist(idx4, init_deg, ones_chunk)
  g1, dis3 = _scale_split(deg, mm1)
  acc1 = _aggregate(g1, idx4)
  g2 = _mid_dense(acc1, dis3, conv1_bias, conv2_weight)
  acc2 = _aggregate_edge_split(g2, zeros_feat, idx4)
  return _final(acc2, dis3, conv2_bias)
